# Initial kernel scaffold; baseline (speedup 1.0000x reference)
import jax
import jax.numpy as jnp
from jax.experimental import pallas as pl


def kernel(x, edge_index, batch, W1, b1, W2, b2, W3, b3, fW1, fb1, fW2, fb2):
    raise NotImplementedError("write your pallas kernel here")


# speedup vs baseline: 16.8130x; 16.8130x over previous
import functools

import jax
import jax.numpy as jnp
from jax import lax
from jax.experimental import pallas as pl
from jax.experimental.pallas import tpu as pltpu
from jax.experimental.pallas import tpu_sc as plsc

N = 10000
E = 320000
D = 128
H = 128
O_DIM = 8
G = 16

NC = 2
NS = 16
NW = NC * NS
EPW = E // NW
CHUNK = 80
NCHUNK = EPW // CHUNK
NPAD = 10240
RPT = NPAD // NS
WB = RPT // CHUNK

_SC_MESH = plsc.VectorSubcoreMesh(core_axis_name="c", subcore_axis_name="s")


def _hist_body(dst3_hbm, ones_hbm, zcol_hbm, out_hbm, ids_v, ones_v, tmp_v,
               hist_sp):
    c = lax.axis_index("c")
    s = lax.axis_index("s")
    wid = c * NS + s
    pltpu.sync_copy(dst3_hbm.at[wid], ids_v)
    pltpu.sync_copy(ones_hbm, ones_v)
    pltpu.sync_copy(zcol_hbm, hist_sp.at[pl.ds(s * RPT, RPT)])
    plsc.subcore_barrier()

    def body(k, carry):
        pltpu.sync_copy(ones_v, hist_sp.at[ids_v.at[k]], add=True)
        return carry

    lax.fori_loop(0, NCHUNK, body, 0)
    plsc.subcore_barrier()
    pltpu.sync_copy(hist_sp.at[pl.ds(s * RPT, RPT)], tmp_v)
    pltpu.sync_copy(tmp_v, out_hbm.at[pl.ds(c * NPAD + s * RPT, RPT)])


@functools.partial(
    pl.kernel,
    out_type=jax.ShapeDtypeStruct((NC * NPAD,), jnp.float32),
    mesh=_SC_MESH,
    scratch_types=[
        pltpu.VMEM((NCHUNK, CHUNK), jnp.int32),
        pltpu.VMEM((CHUNK,), jnp.float32),
        pltpu.VMEM((RPT,), jnp.float32),
        pltpu.VMEM_SHARED((NPAD,), jnp.float32),
    ],
)
def _sc_hist(dst3_hbm, ones_hbm, zcol_hbm, out_hbm, ids_v, ones_v, tmp_v,
             hist_sp):
    _hist_body(dst3_hbm, ones_hbm, zcol_hbm, out_hbm, ids_v, ones_v, tmp_v,
               hist_sp)


def _agg_body(u_hbm, src3_hbm, dst3_hbm, zrows_hbm, out_hbm, srcs_v, dsts_v,
              rows_v, sem, acc_sp):
    c = lax.axis_index("c")
    s = lax.axis_index("s")
    wid = c * NS + s
    pltpu.sync_copy(src3_hbm.at[wid], srcs_v)
    pltpu.sync_copy(dst3_hbm.at[wid], dsts_v)
    pltpu.sync_copy(zrows_hbm, rows_v)
    for j in range(WB):
        pltpu.sync_copy(rows_v, acc_sp.at[pl.ds(s * RPT + j * CHUNK, CHUNK)])
    plsc.subcore_barrier()

    def body(k, carry):
        pltpu.async_copy(u_hbm.at[srcs_v.at[k]], rows_v, sem).wait()
        pltpu.sync_copy(rows_v, acc_sp.at[dsts_v.at[k]], add=True)
        return carry

    lax.fori_loop(0, NCHUNK, body, 0)
    plsc.subcore_barrier()
    for j in range(WB):
        r0 = s * RPT + j * CHUNK
        pltpu.sync_copy(acc_sp.at[pl.ds(r0, CHUNK)], rows_v)
        pltpu.sync_copy(rows_v, out_hbm.at[pl.ds(c * NPAD + r0, CHUNK)])


@functools.partial(
    pl.kernel,
    out_type=jax.ShapeDtypeStruct((NC * NPAD, H), jnp.float32),
    mesh=_SC_MESH,
    scratch_types=[
        pltpu.VMEM((NCHUNK, CHUNK), jnp.int32),
        pltpu.VMEM((NCHUNK, CHUNK), jnp.int32),
        pltpu.VMEM((CHUNK, H), jnp.float32),
        pltpu.SemaphoreType.DMA,
        pltpu.VMEM_SHARED((NPAD, H), jnp.float32),
    ],
)
def _sc_agg(u_hbm, src3_hbm, dst3_hbm, zrows_hbm, out_hbm, srcs_v, dsts_v,
            rows_v, sem, acc_sp):
    _agg_body(u_hbm, src3_hbm, dst3_hbm, zrows_hbm, out_hbm, srcs_v, dsts_v,
              rows_v, sem, acc_sp)


RB = 1000
NRB = N // RB


def _tc1_body(degt_ref, x_ref, w_ref, u_ref, dinv_ref):
    deg = jnp.sum(degt_ref[...], axis=1, keepdims=True) + 1.0
    dinv = lax.rsqrt(deg)
    dinv_ref[...] = dinv
    hw = jnp.dot(x_ref[...], w_ref[...], preferred_element_type=jnp.float32)
    u_ref[...] = hw * dinv


def _tc_first(degt, x, w):
    return pl.pallas_call(
        _tc1_body,
        grid=(NRB,),
        in_specs=[
            pl.BlockSpec((RB, 2), lambda i: (i, 0)),
            pl.BlockSpec((RB, D), lambda i: (i, 0)),
            pl.BlockSpec((D, H), lambda i: (0, 0)),
        ],
        out_specs=[
            pl.BlockSpec((RB, H), lambda i: (i, 0)),
            pl.BlockSpec((RB, 1), lambda i: (i, 0)),
        ],
        out_shape=[
            jax.ShapeDtypeStruct((N, H), jnp.float32),
            jax.ShapeDtypeStruct((N, 1), jnp.float32),
        ],
    )(degt, x, w)


def _tc_mid_body(acc_ref, u_ref, dinv_ref, b_ref, w_ref, un_ref):
    dinv = dinv_ref[...]
    h = acc_ref[0] + acc_ref[1] + u_ref[...]
    h = jnp.maximum(h * dinv + b_ref[...], 0.0)
    hw = jnp.dot(h, w_ref[...], preferred_element_type=jnp.float32)
    un_ref[...] = hw * dinv


def _tc_mid(acc, u, dinv, b, w):
    return pl.pallas_call(
        _tc_mid_body,
        grid=(NRB,),
        in_specs=[
            pl.BlockSpec((NC, RB, H), lambda i: (0, i, 0)),
            pl.BlockSpec((RB, H), lambda i: (i, 0)),
            pl.BlockSpec((RB, 1), lambda i: (i, 0)),
            pl.BlockSpec((1, H), lambda i: (0, 0)),
            pl.BlockSpec((H, H), lambda i: (0, 0)),
        ],
        out_specs=pl.BlockSpec((RB, H), lambda i: (i, 0)),
        out_shape=jax.ShapeDtypeStruct((N, H), jnp.float32),
    )(acc, u, dinv, b, w)


def _tc_final_body(acc_ref, u_ref, dinv_ref, b_ref, batch_ref, fw1_ref,
                   fb1_ref, fw2_ref, fb2_ref, out_ref, sums_scr, cnt_scr):
    i = pl.program_id(0)
    h = acc_ref[0] + acc_ref[1] + u_ref[...]
    h = jnp.maximum(h * dinv_ref[...] + b_ref[...], 0.0)
    gids = lax.broadcasted_iota(jnp.int32, (1, G), 1)
    onehot = (batch_ref[...] == gids).astype(jnp.float32)
    psum = lax.dot_general(onehot, h, (((0,), (0,)), ((), ())),
                           preferred_element_type=jnp.float32)
    pcnt = lax.dot_general(onehot, jnp.ones_like(h), (((0,), (0,)), ((), ())),
                           preferred_element_type=jnp.float32)

    @pl.when(i == 0)
    def _init():
        sums_scr[...] = jnp.zeros_like(sums_scr)
        cnt_scr[...] = jnp.zeros_like(cnt_scr)

    sums_scr[...] += psum
    cnt_scr[...] += pcnt

    @pl.when(i == NRB - 1)
    def _head():
        pooled = sums_scr[...] / jnp.maximum(cnt_scr[...], 1.0)
        hm = jnp.maximum(
            jnp.dot(pooled, fw1_ref[...], preferred_element_type=jnp.float32)
            + fb1_ref[...], 0.0)
        out_ref[...] = (
            jnp.dot(hm, fw2_ref[...], preferred_element_type=jnp.float32)
            + fb2_ref[...])


def _tc_final(acc, u, dinv, b, batch_col, fw1, fb1, fw2, fb2):
    return pl.pallas_call(
        _tc_final_body,
        grid=(NRB,),
        in_specs=[
            pl.BlockSpec((NC, RB, H), lambda i: (0, i, 0)),
            pl.BlockSpec((RB, H), lambda i: (i, 0)),
            pl.BlockSpec((RB, 1), lambda i: (i, 0)),
            pl.BlockSpec((1, H), lambda i: (0, 0)),
            pl.BlockSpec((RB, 1), lambda i: (i, 0)),
            pl.BlockSpec((H, H), lambda i: (0, 0)),
            pl.BlockSpec((1, H), lambda i: (0, 0)),
            pl.BlockSpec((H, O_DIM), lambda i: (0, 0)),
            pl.BlockSpec((1, O_DIM), lambda i: (0, 0)),
        ],
        out_specs=pl.BlockSpec((G, O_DIM), lambda i: (0, 0)),
        out_shape=jax.ShapeDtypeStruct((G, O_DIM), jnp.float32),
        scratch_shapes=[
            pltpu.VMEM((G, H), jnp.float32),
            pltpu.VMEM((G, H), jnp.float32),
        ],
    )(acc, u, dinv, b, batch_col, fw1, fb1, fw2, fb2)


def kernel(x, edge_index, batch, W1, b1, W2, b2, W3, b3, fW1, fb1, fW2, fb2):
    src3 = edge_index[0].reshape(NW, NCHUNK, CHUNK)
    dst3 = edge_index[1].reshape(NW, NCHUNK, CHUNK)
    ones80 = jnp.ones((CHUNK,), jnp.float32)
    zcol = jnp.zeros((RPT,), jnp.float32)
    zrows = jnp.zeros((CHUNK, H), jnp.float32)
    batch_col = batch.reshape(N, 1)

    degflat = _sc_hist(dst3, ones80, zcol)
    degt = degflat.reshape(NC, NPAD).T

    u1, dinv = _tc_first(degt[:N], x, W1)
    acc1 = _sc_agg(u1, src3, dst3, zrows).reshape(NC, NPAD, H)
    u2 = _tc_mid(acc1, u1, dinv, b1.reshape(1, H), W2)
    acc2 = _sc_agg(u2, src3, dst3, zrows).reshape(NC, NPAD, H)
    u3 = _tc_mid(acc2, u2, dinv, b2.reshape(1, H), W3)
    acc3 = _sc_agg(u3, src3, dst3, zrows).reshape(NC, NPAD, H)
    out = _tc_final(acc3, u3, dinv, b3.reshape(1, H), batch_col,
                    fW1, fb1.reshape(1, H), fW2, fb2.reshape(1, O_DIM))
    return out


# speedup vs baseline: 18.5012x; 1.1004x over previous
import functools

import jax
import jax.numpy as jnp
from jax import lax
from jax.experimental import pallas as pl
from jax.experimental.pallas import tpu as pltpu
from jax.experimental.pallas import tpu_sc as plsc

N = 10000
E = 320000
D = 128
H = 128
O_DIM = 8
G = 16

NC = 2
NS = 16
NW = NC * NS
CHUNK = 64
NCHUNK = 157
EPW = NCHUNK * CHUNK
EPAD = NW * EPW
NPAD = 10240
RPT = NPAD // NS
WB = RPT // CHUNK

_SC_MESH = plsc.VectorSubcoreMesh(core_axis_name="c", subcore_axis_name="s")


def _unpack_chunk(packed_v, stage_v, k, src_row, dst_row):
    for j in range(CHUNK // 16):
        p = packed_v[k, pl.ds(j * 16, 16)]
        stage_v[src_row, pl.ds(j * 16, 16)] = lax.bitwise_and(p, 0xFFFF)
        stage_v[dst_row, pl.ds(j * 16, 16)] = lax.shift_right_logical(p, 16)


def _hist_body(pk3_hbm, ones_hbm, zcol_hbm, out_hbm, packed_v, stage_v,
               ones_v, tmp_v, hist_sp):
    c = lax.axis_index("c")
    s = lax.axis_index("s")
    wid = c * NS + s
    pltpu.sync_copy(pk3_hbm.at[wid], packed_v)
    pltpu.sync_copy(ones_hbm, ones_v)
    pltpu.sync_copy(zcol_hbm, hist_sp.at[pl.ds(s * RPT, RPT)])
    plsc.subcore_barrier()

    def body(k, carry):
        _unpack_chunk(packed_v, stage_v, k, 0, 1)
        pltpu.sync_copy(ones_v, hist_sp.at[stage_v.at[1]], add=True)
        return carry

    lax.fori_loop(0, NCHUNK, body, 0)
    plsc.subcore_barrier()
    pltpu.sync_copy(hist_sp.at[pl.ds(s * RPT, RPT)], tmp_v)
    pltpu.sync_copy(tmp_v, out_hbm.at[pl.ds(c * NPAD + s * RPT, RPT)])


@functools.partial(
    pl.kernel,
    out_type=jax.ShapeDtypeStruct((NC * NPAD,), jnp.float32),
    mesh=_SC_MESH,
    scratch_types=[
        pltpu.VMEM((NCHUNK, CHUNK), jnp.int32),
        pltpu.VMEM((4, CHUNK), jnp.int32),
        pltpu.VMEM((CHUNK,), jnp.float32),
        pltpu.VMEM((RPT,), jnp.float32),
        pltpu.VMEM_SHARED((NPAD,), jnp.float32),
    ],
)
def _sc_hist(pk3_hbm, ones_hbm, zcol_hbm, out_hbm, packed_v, stage_v, ones_v,
             tmp_v, hist_sp):
    _hist_body(pk3_hbm, ones_hbm, zcol_hbm, out_hbm, packed_v, stage_v,
               ones_v, tmp_v, hist_sp)


def _agg_body(u_hbm, pk3_hbm, zrows_hbm, out_hbm, packed_v, stage_v,
              rows0_v, rows1_v, sem0, sem1, acc_sp):
    c = lax.axis_index("c")
    s = lax.axis_index("s")
    wid = c * NS + s
    pltpu.sync_copy(pk3_hbm.at[wid], packed_v)
    pltpu.sync_copy(zrows_hbm, rows0_v)
    for j in range(WB):
        pltpu.sync_copy(rows0_v, acc_sp.at[pl.ds(s * RPT + j * CHUNK, CHUNK)])
    plsc.subcore_barrier()

    _unpack_chunk(packed_v, stage_v, 0, 0, 1)
    pltpu.async_copy(u_hbm.at[stage_v.at[0]], rows0_v, sem0)

    def body(k2, carry):
        a = 2 * k2
        _unpack_chunk(packed_v, stage_v, a + 1, 2, 3)
        pltpu.async_copy(u_hbm.at[stage_v.at[2]], rows1_v, sem1)
        pltpu.make_async_copy(u_hbm.at[stage_v.at[0]], rows0_v, sem0).wait()
        pltpu.sync_copy(rows0_v, acc_sp.at[stage_v.at[1]], add=True)
        _unpack_chunk(packed_v, stage_v, a + 2, 0, 1)
        pltpu.async_copy(u_hbm.at[stage_v.at[0]], rows0_v, sem0)
        pltpu.make_async_copy(u_hbm.at[stage_v.at[2]], rows1_v, sem1).wait()
        pltpu.sync_copy(rows1_v, acc_sp.at[stage_v.at[3]], add=True)
        return carry

    lax.fori_loop(0, (NCHUNK - 1) // 2, body, 0)
    pltpu.make_async_copy(u_hbm.at[stage_v.at[0]], rows0_v, sem0).wait()
    pltpu.sync_copy(rows0_v, acc_sp.at[stage_v.at[1]], add=True)
    plsc.subcore_barrier()
    for j in range(WB):
        r0 = s * RPT + j * CHUNK
        pltpu.sync_copy(acc_sp.at[pl.ds(r0, CHUNK)], rows0_v)
        pltpu.sync_copy(rows0_v, out_hbm.at[pl.ds(c * NPAD + r0, CHUNK)])


@functools.partial(
    pl.kernel,
    out_type=jax.ShapeDtypeStruct((NC * NPAD, H), jnp.float32),
    mesh=_SC_MESH,
    scratch_types=[
        pltpu.VMEM((NCHUNK, CHUNK), jnp.int32),
        pltpu.VMEM((4, CHUNK), jnp.int32),
        pltpu.VMEM((CHUNK, H), jnp.float32),
        pltpu.VMEM((CHUNK, H), jnp.float32),
        pltpu.SemaphoreType.DMA,
        pltpu.SemaphoreType.DMA,
        pltpu.VMEM_SHARED((NPAD, H), jnp.float32),
    ],
)
def _sc_agg(u_hbm, pk3_hbm, zrows_hbm, out_hbm, packed_v, stage_v,
            rows0_v, rows1_v, sem0, sem1, acc_sp):
    _agg_body(u_hbm, pk3_hbm, zrows_hbm, out_hbm, packed_v, stage_v,
              rows0_v, rows1_v, sem0, sem1, acc_sp)


RB = 1000
NRB = N // RB


def _tc1_body(degt_ref, x_ref, w_ref, u_ref, dinv_ref):
    deg = jnp.sum(degt_ref[...], axis=1, keepdims=True) + 1.0
    dinv = lax.rsqrt(deg)
    dinv_ref[...] = dinv
    hw = jnp.dot(x_ref[...], w_ref[...], preferred_element_type=jnp.float32)
    u_ref[...] = hw * dinv


def _tc_first(degt, x, w):
    return pl.pallas_call(
        _tc1_body,
        grid=(NRB,),
        in_specs=[
            pl.BlockSpec((RB, 2), lambda i: (i, 0)),
            pl.BlockSpec((RB, D), lambda i: (i, 0)),
            pl.BlockSpec((D, H), lambda i: (0, 0)),
        ],
        out_specs=[
            pl.BlockSpec((RB, H), lambda i: (i, 0)),
            pl.BlockSpec((RB, 1), lambda i: (i, 0)),
        ],
        out_shape=[
            jax.ShapeDtypeStruct((N, H), jnp.float32),
            jax.ShapeDtypeStruct((N, 1), jnp.float32),
        ],
    )(degt, x, w)


def _tc_mid_body(acc_ref, u_ref, dinv_ref, b_ref, w_ref, un_ref):
    dinv = dinv_ref[...]
    h = acc_ref[0] + acc_ref[1] + u_ref[...]
    h = jnp.maximum(h * dinv + b_ref[...], 0.0)
    hw = jnp.dot(h, w_ref[...], preferred_element_type=jnp.float32)
    un_ref[...] = hw * dinv


def _tc_mid(acc, u, dinv, b, w):
    return pl.pallas_call(
        _tc_mid_body,
        grid=(NRB,),
        in_specs=[
            pl.BlockSpec((NC, RB, H), lambda i: (0, i, 0)),
            pl.BlockSpec((RB, H), lambda i: (i, 0)),
            pl.BlockSpec((RB, 1), lambda i: (i, 0)),
            pl.BlockSpec((1, H), lambda i: (0, 0)),
            pl.BlockSpec((H, H), lambda i: (0, 0)),
        ],
        out_specs=pl.BlockSpec((RB, H), lambda i: (i, 0)),
        out_shape=jax.ShapeDtypeStruct((N, H), jnp.float32),
    )(acc, u, dinv, b, w)


def _tc_final_body(acc_ref, u_ref, dinv_ref, b_ref, batch_ref, fw1_ref,
                   fb1_ref, fw2_ref, fb2_ref, out_ref, sums_scr, cnt_scr):
    i = pl.program_id(0)
    h = acc_ref[0] + acc_ref[1] + u_ref[...]
    h = jnp.maximum(h * dinv_ref[...] + b_ref[...], 0.0)
    gids = lax.broadcasted_iota(jnp.int32, (1, G), 1)
    onehot = (batch_ref[...] == gids).astype(jnp.float32)
    psum = lax.dot_general(onehot, h, (((0,), (0,)), ((), ())),
                           preferred_element_type=jnp.float32)
    pcnt = lax.dot_general(onehot, jnp.ones_like(h), (((0,), (0,)), ((), ())),
                           preferred_element_type=jnp.float32)

    @pl.when(i == 0)
    def _init():
        sums_scr[...] = jnp.zeros_like(sums_scr)
        cnt_scr[...] = jnp.zeros_like(cnt_scr)

    sums_scr[...] += psum
    cnt_scr[...] += pcnt

    @pl.when(i == NRB - 1)
    def _head():
        pooled = sums_scr[...] / jnp.maximum(cnt_scr[...], 1.0)
        hm = jnp.maximum(
            jnp.dot(pooled, fw1_ref[...], preferred_element_type=jnp.float32)
            + fb1_ref[...], 0.0)
        out_ref[...] = (
            jnp.dot(hm, fw2_ref[...], preferred_element_type=jnp.float32)
            + fb2_ref[...])


def _tc_final(acc, u, dinv, b, batch_col, fw1, fb1, fw2, fb2):
    return pl.pallas_call(
        _tc_final_body,
        grid=(NRB,),
        in_specs=[
            pl.BlockSpec((NC, RB, H), lambda i: (0, i, 0)),
            pl.BlockSpec((RB, H), lambda i: (i, 0)),
            pl.BlockSpec((RB, 1), lambda i: (i, 0)),
            pl.BlockSpec((1, H), lambda i: (0, 0)),
            pl.BlockSpec((RB, 1), lambda i: (i, 0)),
            pl.BlockSpec((H, H), lambda i: (0, 0)),
            pl.BlockSpec((1, H), lambda i: (0, 0)),
            pl.BlockSpec((H, O_DIM), lambda i: (0, 0)),
            pl.BlockSpec((1, O_DIM), lambda i: (0, 0)),
        ],
        out_specs=pl.BlockSpec((G, O_DIM), lambda i: (0, 0)),
        out_shape=jax.ShapeDtypeStruct((G, O_DIM), jnp.float32),
        scratch_shapes=[
            pltpu.VMEM((G, H), jnp.float32),
            pltpu.VMEM((G, H), jnp.float32),
        ],
    )(acc, u, dinv, b, batch_col, fw1, fb1, fw2, fb2)


def kernel(x, edge_index, batch, W1, b1, W2, b2, W3, b3, fW1, fb1, fW2, fb2):
    npad_e = EPAD - E
    src_p = jnp.concatenate(
        [edge_index[0], jnp.zeros((npad_e,), edge_index.dtype)])
    dst_p = jnp.concatenate(
        [edge_index[1], jnp.full((npad_e,), NPAD - 1, edge_index.dtype)])
    pk3 = (src_p | (dst_p << 16)).reshape(NW, NCHUNK, CHUNK)
    ones80 = jnp.ones((CHUNK,), jnp.float32)
    zcol = jnp.zeros((RPT,), jnp.float32)
    zrows = jnp.zeros((CHUNK, H), jnp.float32)
    batch_col = batch.reshape(N, 1)

    degflat = _sc_hist(pk3, ones80, zcol)
    degt = degflat.reshape(NC, NPAD).T

    u1, dinv = _tc_first(degt[:N], x, W1)
    acc1 = _sc_agg(u1, pk3, zrows).reshape(NC, NPAD, H)
    u2 = _tc_mid(acc1, u1, dinv, b1.reshape(1, H), W2)
    acc2 = _sc_agg(u2, pk3, zrows).reshape(NC, NPAD, H)
    u3 = _tc_mid(acc2, u2, dinv, b2.reshape(1, H), W3)
    acc3 = _sc_agg(u3, pk3, zrows).reshape(NC, NPAD, H)
    out = _tc_final(acc3, u3, dinv, b3.reshape(1, H), batch_col,
                    fW1, fb1.reshape(1, H), fW2, fb2.reshape(1, O_DIM))
    return out
